# baseline (speedup 1.0000x reference)
import functools

import jax
import jax.numpy as jnp
import numpy as np
from jax import lax
from jax.experimental import pallas as pl
from jax.experimental.pallas import tpu as pltpu
from jax.experimental.pallas import tpu_sc as plsc

_NUM_FIELDS = 26
_FIELD_DIM = 100000
_EMBED_DIM = 16
_BATCH = 4096

_NC, _NS, _L = 2, 16, 16
_NW = _NC * _NS
_B = _BATCH * _NUM_FIELDS
_BPW = _B // _NW
_CHUNK = 128
_NCHUNK = _BPW // _CHUNK
_NSLICE = _BPW // _L

_OFFS = np.asarray((np.arange(_BPW) % _NUM_FIELDS) * _FIELD_DIM, np.int32)

_mesh = plsc.VectorSubcoreMesh(
    core_axis_name="c", subcore_axis_name="s", num_cores=_NC, num_subcores=_NS
)


@functools.partial(
    pl.kernel,
    out_type=jax.ShapeDtypeStruct((_B, _EMBED_DIM), jnp.float32),
    mesh=_mesh,
    scratch_types=[
        pltpu.VMEM((_BPW,), jnp.int32),
        pltpu.VMEM((_BPW,), jnp.int32),
        pltpu.VMEM((_BPW, _EMBED_DIM), jnp.float32),
        pltpu.SemaphoreType.DMA,
    ],
    compiler_params=pltpu.CompilerParams(use_tc_tiling_on_sc=False),
)
def _embed_gather(x_hbm, offs_hbm, table_hbm, out_hbm, idx_v, offs_v, rows_v, sem):
    wid = lax.axis_index("s") * _NC + lax.axis_index("c")
    base = wid * _BPW

    pltpu.sync_copy(x_hbm.at[pl.ds(base, _BPW)], idx_v)
    pltpu.sync_copy(offs_hbm, offs_v)

    def add_offsets(i, _):
        s = i * _L
        idx_v[pl.ds(s, _L)] = idx_v[pl.ds(s, _L)] + offs_v[pl.ds(s, _L)]
        return 0

    lax.fori_loop(0, _NSLICE, add_offsets, 0)

    copies = [
        pltpu.async_copy(
            table_hbm.at[idx_v.at[pl.ds(j * _CHUNK, _CHUNK)]],
            rows_v.at[pl.ds(j * _CHUNK, _CHUNK)],
            sem,
        )
        for j in range(_NCHUNK)
    ]
    for cp in copies:
        cp.wait()

    pltpu.sync_copy(rows_v, out_hbm.at[pl.ds(base, _BPW)])


def kernel(x, table):
    x_flat = x.reshape(_B).astype(jnp.int32)
    out = _embed_gather(x_flat, jnp.asarray(_OFFS), table)
    return out.reshape(_BATCH, _NUM_FIELDS, _EMBED_DIM)
